# speedup vs baseline: 1.3445x; 1.3445x over previous
import functools

import jax
import jax.numpy as jnp
import numpy as np
from jax.experimental import pallas as pl

N = 10000
E = 320000
F_IN = 128
HID = 256
C = 40
K = 10

_BN = 1000


def _mlp_body(x_ref, w1t_ref, b1_ref, w2t_ref, b2_ref, o_ref):
    h = jnp.maximum(x_ref[...] @ w1t_ref[...] + b1_ref[...], 0.0)
    o_ref[...] = h @ w2t_ref[...] + b2_ref[...]


def _mlp(x, W1, b1, W2, b2):
    w1t = W1.T
    w2t = W2.T
    b1r = b1.reshape(1, HID)
    b2r = b2.reshape(1, C)
    return pl.pallas_call(
        _mlp_body,
        grid=(N // _BN,),
        in_specs=[
            pl.BlockSpec((_BN, F_IN), lambda i: (i, 0)),
            pl.BlockSpec((F_IN, HID), lambda i: (0, 0)),
            pl.BlockSpec((1, HID), lambda i: (0, 0)),
            pl.BlockSpec((HID, C), lambda i: (0, 0)),
            pl.BlockSpec((1, C), lambda i: (0, 0)),
        ],
        out_specs=pl.BlockSpec((_BN, C), lambda i: (i, 0)),
        out_shape=jax.ShapeDtypeStruct((N, C), jnp.float32),
    )(x, w1t, b1r, w2t, b2r)


def _final_body(pps_ref, pw_ref, pb_ref, o_ref):
    pps = pps_ref[...]
    pw = pw_ref[...]
    logits = jnp.sum(pps * pw[0][None, None, :], axis=-1) + pb_ref[0, 0]
    retain = jax.nn.sigmoid(logits)
    out = jnp.sum(retain[:, :, None] * pps, axis=1)
    m = jnp.max(out, axis=-1, keepdims=True)
    z = out - m
    lse = jnp.log(jnp.sum(jnp.exp(z), axis=-1, keepdims=True))
    o_ref[...] = z - lse


def _final(pps, proj_w, proj_b):
    pbr = proj_b.reshape(1, 1)
    return pl.pallas_call(
        _final_body,
        grid=(N // _BN,),
        in_specs=[
            pl.BlockSpec((_BN, K + 1, C), lambda i: (i, 0, 0)),
            pl.BlockSpec((1, C), lambda i: (0, 0)),
            pl.BlockSpec((1, 1), lambda i: (0, 0)),
        ],
        out_specs=pl.BlockSpec((_BN, C), lambda i: (i, 0)),
        out_shape=jax.ShapeDtypeStruct((N, C), jnp.float32),
    )(pps, proj_w, pbr)


def kernel(x, edge_index, W1, b1, W2, b2, proj_w, proj_b, temp):
    row = edge_index[0]
    col = edge_index[1]
    loop_mask = row == col
    loop_counts = jnp.zeros((N,), jnp.int32).at[col].add(loop_mask.astype(jnp.int32))
    self_w = jnp.where(loop_counts > 0, 0.0, 1.0).astype(jnp.float32)
    deg = jnp.zeros((N,), jnp.float32).at[col].add(jnp.ones((E,), jnp.float32)) + self_w
    dis = jnp.where(deg > 0, deg ** -0.5, 0.0)

    h = _mlp(x, W1, b1, W2, b2)

    norm = dis[row] * dis[col]
    sdiag = self_w * dis * dis

    hidden = h * temp[0]
    preds = [h]
    xk = h
    for k in range(K):
        msg = norm[:, None] * xk[row]
        xk = (jnp.zeros((N, C), jnp.float32).at[col].add(msg)
              + sdiag[:, None] * xk)
        hidden = hidden + temp[k + 1] * xk
        preds.append(hidden)
    pps = jnp.stack(preds, axis=1)
    return _final(pps, proj_w, proj_b)


# speedup vs baseline: 14.0847x; 10.4761x over previous
import functools

import jax
import jax.numpy as jnp
from jax import lax
from jax.experimental import pallas as pl
from jax.experimental.pallas import tpu as pltpu
from jax.experimental.pallas import tpu_sc as plsc

N = 10000
E = 320000
F_IN = 128
HID = 256
C = 40
K = 10

NPAD = 10240
CP = 48
NTILES = 32
EPT = 10240
CH = 128
NCH = EPT // CH
ROWS_PT = NPAD // 16
EPAD = NTILES * EPT

_BN = 1024


def _mlp_body(x_ref, w1t_ref, b1_ref, w2t_ref, b2_ref, d_ref, o_ref):
    h = jnp.maximum(x_ref[...] @ w1t_ref[...] + b1_ref[...], 0.0)
    o_ref[...] = d_ref[...] * (h @ w2t_ref[...] + b2_ref[...])


def _mlp(xp, W1, b1, W2, b2, d):
    w1t = W1.T
    w2t = jnp.pad(W2.T, ((0, 0), (0, CP - C)))
    b1r = b1.reshape(1, HID)
    b2r = jnp.pad(b2.reshape(1, C), ((0, 0), (0, CP - C)))
    return pl.pallas_call(
        _mlp_body,
        grid=(NPAD // _BN,),
        in_specs=[
            pl.BlockSpec((_BN, F_IN), lambda i: (i, 0)),
            pl.BlockSpec((F_IN, HID), lambda i: (0, 0)),
            pl.BlockSpec((1, HID), lambda i: (0, 0)),
            pl.BlockSpec((HID, CP), lambda i: (0, 0)),
            pl.BlockSpec((1, CP), lambda i: (0, 0)),
            pl.BlockSpec((_BN, 1), lambda i: (i, 0)),
        ],
        out_specs=pl.BlockSpec((_BN, CP), lambda i: (i, 0)),
        out_shape=jax.ShapeDtypeStruct((NPAD, CP), jnp.float32),
    )(xp, w1t, b1r, w2t, b2r, d)


def _make_sc_step():
    mesh = plsc.VectorSubcoreMesh(core_axis_name="c", subcore_axis_name="s")

    @functools.partial(
        pl.kernel,
        mesh=mesh,
        compiler_params=pltpu.CompilerParams(use_tc_tiling_on_sc=False),
        out_type=jax.ShapeDtypeStruct((2 * NPAD, CP), jnp.float32),
        scratch_types=[
            pltpu.VMEM((NCH, CH), jnp.int32),
            pltpu.VMEM((NCH, CH), jnp.int32),
            pltpu.VMEM((CH, CP), jnp.float32),
            pltpu.VMEM_SHARED((NPAD, CP), jnp.float32),
            pltpu.SemaphoreType.DMA,
        ],
    )
    def step(u_hbm, rowv_hbm, colv_hbm, zeros_hbm, p_out,
             row_v, col_v, msg, acc, sem):
        cid = lax.axis_index("c")
        sid = lax.axis_index("s")
        wid = sid * 2 + cid
        pltpu.sync_copy(rowv_hbm.at[wid], row_v)
        pltpu.sync_copy(colv_hbm.at[wid], col_v)
        pltpu.sync_copy(zeros_hbm.at[pl.ds(sid * ROWS_PT, ROWS_PT)],
                        acc.at[pl.ds(sid * ROWS_PT, ROWS_PT)])
        plsc.subcore_barrier()

        def body(j, carry):
            pltpu.async_copy(u_hbm.at[row_v.at[j]], msg, sem).wait()
            pltpu.sync_copy(msg, acc.at[col_v.at[j]], add=True)
            return carry

        lax.fori_loop(0, NCH, body, 0)
        plsc.subcore_barrier()
        pltpu.sync_copy(acc.at[pl.ds(sid * ROWS_PT, ROWS_PT)],
                        p_out.at[pl.ds(cid * NPAD + sid * ROWS_PT, ROWS_PT)])

    return step


def _combine_body(p_ref, u_ref, e_ref, g_ref, o_ref):
    o_ref[...] = (e_ref[...] * (p_ref[0] + p_ref[1])
                  + g_ref[...] * u_ref[...])


def _combine(p, u, e, g):
    return pl.pallas_call(
        _combine_body,
        grid=(NPAD // _BN,),
        in_specs=[
            pl.BlockSpec((2, _BN, CP), lambda i: (0, i, 0)),
            pl.BlockSpec((_BN, CP), lambda i: (i, 0)),
            pl.BlockSpec((_BN, 1), lambda i: (i, 0)),
            pl.BlockSpec((_BN, 1), lambda i: (i, 0)),
        ],
        out_specs=pl.BlockSpec((_BN, CP), lambda i: (i, 0)),
        out_shape=jax.ShapeDtypeStruct((NPAD, CP), jnp.float32),
    )(p.reshape(2, NPAD, CP), u, e, g)


def _final_body(u_ref, temp_ref, dinv_ref, pw_ref, pb_ref, o_ref):
    dinv = dinv_ref[...]
    pw = pw_ref[...]
    pb = pb_ref[0, 0]
    bn = dinv.shape[0]

    out = jnp.zeros((bn, CP), jnp.float32)
    running = temp_ref[0, 0] * u_ref[0]
    for j in range(K + 1):
        if j == 0:
            pj = dinv * u_ref[0]
        else:
            running = running + temp_ref[0, j] * u_ref[j]
            pj = dinv * running
        logit = jnp.sum(pj * pw, axis=-1, keepdims=True) + pb
        out = out + jax.nn.sigmoid(logit) * pj

    mask = lax.broadcasted_iota(jnp.int32, (bn, CP), 1) < C
    m = jnp.max(jnp.where(mask, out, -1e30), axis=-1, keepdims=True)
    z = out - m
    ez = jnp.where(mask, jnp.exp(z), 0.0)
    lse = jnp.log(jnp.sum(ez, axis=-1, keepdims=True))
    o_ref[...] = z - lse


def _final(U, temp, dinv, proj_w, proj_b):
    tempp = jnp.pad(temp.reshape(1, K + 1), ((0, 0), (0, 16 - (K + 1))))
    pwp = jnp.pad(proj_w, ((0, 0), (0, CP - C)))
    pbr = proj_b.reshape(1, 1)
    return pl.pallas_call(
        _final_body,
        grid=(NPAD // _BN,),
        in_specs=[
            pl.BlockSpec((K + 1, _BN, CP), lambda i: (0, i, 0)),
            pl.BlockSpec((1, 16), lambda i: (0, 0)),
            pl.BlockSpec((_BN, 1), lambda i: (i, 0)),
            pl.BlockSpec((1, CP), lambda i: (0, 0)),
            pl.BlockSpec((1, 1), lambda i: (0, 0)),
        ],
        out_specs=pl.BlockSpec((_BN, CP), lambda i: (i, 0)),
        out_shape=jax.ShapeDtypeStruct((NPAD, CP), jnp.float32),
    )(U, tempp, dinv, pwp, pbr)


def kernel(x, edge_index, W1, b1, W2, b2, proj_w, proj_b, temp):
    row = edge_index[0]
    col = edge_index[1]

    loop_mask = row == col
    loop_counts = jnp.zeros((N,), jnp.int32).at[col].add(loop_mask.astype(jnp.int32))
    self_w = jnp.where(loop_counts > 0, 0.0, 1.0).astype(jnp.float32)
    deg = jnp.zeros((N,), jnp.float32).at[col].add(jnp.ones((E,), jnp.float32)) + self_w

    zpad = jnp.zeros((NPAD - N,), jnp.float32)
    e = jnp.concatenate([1.0 / deg, zpad]).reshape(NPAD, 1)
    g = jnp.concatenate([self_w / deg, zpad]).reshape(NPAD, 1)
    d = jnp.concatenate([deg ** -0.5, zpad]).reshape(NPAD, 1)
    dinv = jnp.concatenate([deg ** 0.5, zpad]).reshape(NPAD, 1)

    pad_idx = (N + (jnp.arange(EPAD - E, dtype=jnp.int32) % (NPAD - N))).astype(jnp.int32)
    rowp = jnp.concatenate([row, pad_idx]).reshape(NTILES, NCH, CH)
    colp = jnp.concatenate([col, pad_idx]).reshape(NTILES, NCH, CH)
    zeros_nc = jnp.zeros((NPAD, CP), jnp.float32)

    xp = jnp.pad(x, ((0, NPAD - N), (0, 0)))
    u0 = _mlp(xp, W1, b1, W2, b2, d)

    sc_step = _make_sc_step()
    us = [u0]
    u = u0
    for _ in range(K):
        p = sc_step(u, rowp, colp, zeros_nc)
        u = _combine(p, u, e, g)
        us.append(u)

    U = jnp.stack(us, axis=0)
    out = _final(U, temp, dinv, proj_w, proj_b)
    return out[:N, :C]


# speedup vs baseline: 18.5710x; 1.3185x over previous
import functools

import jax
import jax.numpy as jnp
from jax import lax
from jax.experimental import pallas as pl
from jax.experimental.pallas import tpu as pltpu
from jax.experimental.pallas import tpu_sc as plsc

N = 10000
E = 320000
F_IN = 128
HID = 256
C = 40
K = 10

NPAD = 10240
CP = 48
NTILES = 32
EPT = 10240
CH = 128
NCH = EPT // CH
ROWS_PT = NPAD // 16
EPAD = NTILES * EPT

_BN = 1024


def _mlp_body(x_ref, w1t_ref, b1_ref, w2t_ref, b2_ref, d_ref, o_ref):
    h = jnp.maximum(x_ref[...] @ w1t_ref[...] + b1_ref[...], 0.0)
    o_ref[...] = d_ref[...] * (h @ w2t_ref[...] + b2_ref[...])


def _mlp(xp, W1, b1, W2, b2, d):
    w1t = W1.T
    w2t = jnp.pad(W2.T, ((0, 0), (0, CP - C)))
    b1r = b1.reshape(1, HID)
    b2r = jnp.pad(b2.reshape(1, C), ((0, 0), (0, CP - C)))
    return pl.pallas_call(
        _mlp_body,
        grid=(NPAD // _BN,),
        in_specs=[
            pl.BlockSpec((_BN, F_IN), lambda i: (i, 0)),
            pl.BlockSpec((F_IN, HID), lambda i: (0, 0)),
            pl.BlockSpec((1, HID), lambda i: (0, 0)),
            pl.BlockSpec((HID, CP), lambda i: (0, 0)),
            pl.BlockSpec((1, CP), lambda i: (0, 0)),
            pl.BlockSpec((_BN, 1), lambda i: (i, 0)),
        ],
        out_specs=pl.BlockSpec((_BN, CP), lambda i: (i, 0)),
        out_shape=jax.ShapeDtypeStruct((NPAD, CP), jnp.float32),
    )(xp, w1t, b1r, w2t, b2r, d)


def _make_sc_step():
    mesh = plsc.VectorSubcoreMesh(core_axis_name="c", subcore_axis_name="s")

    @functools.partial(
        pl.kernel,
        mesh=mesh,
        compiler_params=pltpu.CompilerParams(use_tc_tiling_on_sc=False),
        out_type=jax.ShapeDtypeStruct((2 * NPAD, CP), jnp.float32),
        scratch_types=[
            pltpu.VMEM((NCH, CH), jnp.int32),
            pltpu.VMEM((NCH, CH), jnp.int32),
            pltpu.VMEM((CH, CP), jnp.float32),
            pltpu.VMEM((CH, CP), jnp.float32),
            pltpu.VMEM_SHARED((NPAD, CP), jnp.float32),
            pltpu.SemaphoreType.DMA,
            pltpu.SemaphoreType.DMA,
        ],
    )
    def step(u_hbm, rowv_hbm, colv_hbm, zeros_hbm, p_out,
             row_v, col_v, msg0, msg1, acc, sem0, sem1):
        cid = lax.axis_index("c")
        sid = lax.axis_index("s")
        wid = sid * 2 + cid
        pltpu.sync_copy(rowv_hbm.at[wid], row_v)
        pltpu.sync_copy(colv_hbm.at[wid], col_v)
        pltpu.sync_copy(zeros_hbm.at[pl.ds(sid * ROWS_PT, ROWS_PT)],
                        acc.at[pl.ds(sid * ROWS_PT, ROWS_PT)])
        plsc.subcore_barrier()

        pltpu.async_copy(u_hbm.at[row_v.at[0]], msg0, sem0)

        def body(g, carry):
            j0 = 2 * g
            j1 = j0 + 1
            j2 = j0 + 2
            pltpu.async_copy(u_hbm.at[row_v.at[j1]], msg1, sem1)
            pltpu.make_async_copy(u_hbm.at[row_v.at[j0]], msg0, sem0).wait()
            pltpu.sync_copy(msg0, acc.at[col_v.at[j0]], add=True)

            @pl.when(j2 < NCH)
            def _():
                pltpu.async_copy(u_hbm.at[row_v.at[j2]], msg0, sem0)

            pltpu.make_async_copy(u_hbm.at[row_v.at[j1]], msg1, sem1).wait()
            pltpu.sync_copy(msg1, acc.at[col_v.at[j1]], add=True)
            return carry

        lax.fori_loop(0, NCH // 2, body, 0)
        plsc.subcore_barrier()
        pltpu.sync_copy(acc.at[pl.ds(sid * ROWS_PT, ROWS_PT)],
                        p_out.at[pl.ds(cid * NPAD + sid * ROWS_PT, ROWS_PT)])

    return step


def _combine_body(p_ref, u_ref, e_ref, g_ref, o_ref):
    o_ref[...] = (e_ref[...] * (p_ref[0] + p_ref[1])
                  + g_ref[...] * u_ref[...])


def _combine(p, u, e, g):
    return pl.pallas_call(
        _combine_body,
        grid=(NPAD // _BN,),
        in_specs=[
            pl.BlockSpec((2, _BN, CP), lambda i: (0, i, 0)),
            pl.BlockSpec((_BN, CP), lambda i: (i, 0)),
            pl.BlockSpec((_BN, 1), lambda i: (i, 0)),
            pl.BlockSpec((_BN, 1), lambda i: (i, 0)),
        ],
        out_specs=pl.BlockSpec((_BN, CP), lambda i: (i, 0)),
        out_shape=jax.ShapeDtypeStruct((NPAD, CP), jnp.float32),
    )(p.reshape(2, NPAD, CP), u, e, g)


def _final_body(u_ref, temp_ref, dinv_ref, pw_ref, pb_ref, o_ref):
    dinv = dinv_ref[...]
    pw = pw_ref[...]
    pb = pb_ref[0, 0]
    bn = dinv.shape[0]

    out = jnp.zeros((bn, CP), jnp.float32)
    running = temp_ref[0, 0] * u_ref[0]
    for j in range(K + 1):
        if j == 0:
            pj = dinv * u_ref[0]
        else:
            running = running + temp_ref[0, j] * u_ref[j]
            pj = dinv * running
        logit = jnp.sum(pj * pw, axis=-1, keepdims=True) + pb
        out = out + jax.nn.sigmoid(logit) * pj

    mask = lax.broadcasted_iota(jnp.int32, (bn, CP), 1) < C
    m = jnp.max(jnp.where(mask, out, -1e30), axis=-1, keepdims=True)
    z = out - m
    ez = jnp.where(mask, jnp.exp(z), 0.0)
    lse = jnp.log(jnp.sum(ez, axis=-1, keepdims=True))
    o_ref[...] = z - lse


def _final(U, temp, dinv, proj_w, proj_b):
    tempp = jnp.pad(temp.reshape(1, K + 1), ((0, 0), (0, 16 - (K + 1))))
    pwp = jnp.pad(proj_w, ((0, 0), (0, CP - C)))
    pbr = proj_b.reshape(1, 1)
    return pl.pallas_call(
        _final_body,
        grid=(NPAD // _BN,),
        in_specs=[
            pl.BlockSpec((K + 1, _BN, CP), lambda i: (0, i, 0)),
            pl.BlockSpec((1, 16), lambda i: (0, 0)),
            pl.BlockSpec((_BN, 1), lambda i: (i, 0)),
            pl.BlockSpec((1, CP), lambda i: (0, 0)),
            pl.BlockSpec((1, 1), lambda i: (0, 0)),
        ],
        out_specs=pl.BlockSpec((_BN, CP), lambda i: (i, 0)),
        out_shape=jax.ShapeDtypeStruct((NPAD, CP), jnp.float32),
    )(U, tempp, dinv, pwp, pbr)


def kernel(x, edge_index, W1, b1, W2, b2, proj_w, proj_b, temp):
    row = edge_index[0]
    col = edge_index[1]

    loop_mask = row == col
    loop_counts = jnp.zeros((N,), jnp.int32).at[col].add(loop_mask.astype(jnp.int32))
    self_w = jnp.where(loop_counts > 0, 0.0, 1.0).astype(jnp.float32)
    deg = jnp.zeros((N,), jnp.float32).at[col].add(jnp.ones((E,), jnp.float32)) + self_w

    zpad = jnp.zeros((NPAD - N,), jnp.float32)
    e = jnp.concatenate([1.0 / deg, zpad]).reshape(NPAD, 1)
    g = jnp.concatenate([self_w / deg, zpad]).reshape(NPAD, 1)
    d = jnp.concatenate([deg ** -0.5, zpad]).reshape(NPAD, 1)
    dinv = jnp.concatenate([deg ** 0.5, zpad]).reshape(NPAD, 1)

    pad_idx = (N + (jnp.arange(EPAD - E, dtype=jnp.int32) % (NPAD - N))).astype(jnp.int32)
    rowp = jnp.concatenate([row, pad_idx]).reshape(NTILES, NCH, CH)
    colp = jnp.concatenate([col, pad_idx]).reshape(NTILES, NCH, CH)
    zeros_nc = jnp.zeros((NPAD, CP), jnp.float32)

    xp = jnp.pad(x, ((0, NPAD - N), (0, 0)))
    u0 = _mlp(xp, W1, b1, W2, b2, d)

    sc_step = _make_sc_step()
    us = [u0]
    u = u0
    for _ in range(K):
        p = sc_step(u, rowp, colp, zeros_nc)
        u = _combine(p, u, e, g)
        us.append(u)

    U = jnp.stack(us, axis=0)
    out = _final(U, temp, dinv, proj_w, proj_b)
    return out[:N, :C]


# speedup vs baseline: 19.8604x; 1.0694x over previous
import functools

import jax
import jax.numpy as jnp
from jax import lax
from jax.experimental import pallas as pl
from jax.experimental.pallas import tpu as pltpu
from jax.experimental.pallas import tpu_sc as plsc

N = 10000
E = 320000
F_IN = 128
HID = 256
C = 40
K = 10

NPAD = 10240
CP = 48
NTILES = 32
EPT = 10240
CH = 128
NCH = EPT // CH
ROWS_PT = NPAD // 16
EPAD = NTILES * EPT

_BN = 1024


def _mlp_body(x_ref, w1t_ref, b1_ref, w2t_ref, b2_ref, d_ref, o_ref):
    h = jnp.maximum(x_ref[...] @ w1t_ref[...] + b1_ref[...], 0.0)
    o_ref[...] = d_ref[...] * (h @ w2t_ref[...] + b2_ref[...])


def _mlp(xp, W1, b1, W2, b2, d):
    w1t = W1.T
    w2t = jnp.pad(W2.T, ((0, 0), (0, CP - C)))
    b1r = b1.reshape(1, HID)
    b2r = jnp.pad(b2.reshape(1, C), ((0, 0), (0, CP - C)))
    return pl.pallas_call(
        _mlp_body,
        grid=(NPAD // _BN,),
        in_specs=[
            pl.BlockSpec((_BN, F_IN), lambda i: (i, 0)),
            pl.BlockSpec((F_IN, HID), lambda i: (0, 0)),
            pl.BlockSpec((1, HID), lambda i: (0, 0)),
            pl.BlockSpec((HID, CP), lambda i: (0, 0)),
            pl.BlockSpec((1, CP), lambda i: (0, 0)),
            pl.BlockSpec((_BN, 1), lambda i: (i, 0)),
        ],
        out_specs=pl.BlockSpec((_BN, CP), lambda i: (i, 0)),
        out_shape=jax.ShapeDtypeStruct((NPAD, CP), jnp.float32),
    )(xp, w1t, b1r, w2t, b2r, d)


def _make_sc_step():
    mesh = plsc.VectorSubcoreMesh(core_axis_name="c", subcore_axis_name="s")

    @functools.partial(
        pl.kernel,
        mesh=mesh,
        compiler_params=pltpu.CompilerParams(use_tc_tiling_on_sc=False),
        out_type=jax.ShapeDtypeStruct((2 * NPAD, CP), jnp.float32),
        scratch_types=[
            pltpu.VMEM((NCH, CH), jnp.int32),
            pltpu.VMEM((NCH, CH), jnp.int32),
            pltpu.VMEM((CH, CP), jnp.float32),
            pltpu.VMEM((CH, CP), jnp.float32),
            pltpu.VMEM_SHARED((NPAD, CP), jnp.float32),
            pltpu.SemaphoreType.DMA,
            pltpu.SemaphoreType.DMA,
        ],
    )
    def step(u_hbm, rowv_hbm, colv_hbm, zeros_hbm, p_out,
             row_v, col_v, msg0, msg1, acc, sem0, sem1):
        cid = lax.axis_index("c")
        sid = lax.axis_index("s")
        wid = sid * 2 + cid
        pltpu.sync_copy(rowv_hbm.at[wid], row_v)
        pltpu.sync_copy(colv_hbm.at[wid], col_v)
        pltpu.sync_copy(zeros_hbm.at[pl.ds(sid * ROWS_PT, ROWS_PT)],
                        acc.at[pl.ds(sid * ROWS_PT, ROWS_PT)])
        plsc.subcore_barrier()

        pltpu.async_copy(u_hbm.at[row_v.at[0]], msg0, sem0)

        def body(g, carry):
            j0 = 2 * g
            j1 = j0 + 1
            j2 = j0 + 2
            pltpu.async_copy(u_hbm.at[row_v.at[j1]], msg1, sem1)
            pltpu.make_async_copy(u_hbm.at[row_v.at[j0]], msg0, sem0).wait()
            pltpu.sync_copy(msg0, acc.at[col_v.at[j0]], add=True)

            @pl.when(j2 < NCH)
            def _():
                pltpu.async_copy(u_hbm.at[row_v.at[j2]], msg0, sem0)

            pltpu.make_async_copy(u_hbm.at[row_v.at[j1]], msg1, sem1).wait()
            pltpu.sync_copy(msg1, acc.at[col_v.at[j1]], add=True)
            return carry

        lax.fori_loop(0, NCH // 2, body, 0)
        plsc.subcore_barrier()
        pltpu.sync_copy(acc.at[pl.ds(sid * ROWS_PT, ROWS_PT)],
                        p_out.at[pl.ds(cid * NPAD + sid * ROWS_PT, ROWS_PT)])

    return step


CA = 64
NCA = ROWS_PT // CA


def _make_sc_fused_step():
    mesh = plsc.VectorSubcoreMesh(core_axis_name="c", subcore_axis_name="s")

    @functools.partial(
        pl.kernel,
        mesh=mesh,
        compiler_params=pltpu.CompilerParams(use_tc_tiling_on_sc=False),
        out_type=[
            jax.ShapeDtypeStruct((NPAD, CP), jnp.float32),
            jax.ShapeDtypeStruct((2 * NPAD, CP), jnp.float32),
        ],
        scratch_types=[
            pltpu.VMEM((NCH, CH), jnp.int32),
            pltpu.VMEM((NCH, CH), jnp.int32),
            pltpu.VMEM((CH, CP), jnp.float32),
            pltpu.VMEM((CH, CP), jnp.float32),
            pltpu.VMEM((CA, CP), jnp.float32),
            pltpu.VMEM((CA, CP), jnp.float32),
            pltpu.VMEM((CA, CP), jnp.float32),
            pltpu.VMEM((CA, CP), jnp.float32),
            pltpu.VMEM((CA, CP), jnp.float32),
            pltpu.VMEM((CA, CP), jnp.float32),
            pltpu.VMEM_SHARED((NPAD, CP), jnp.float32),
            pltpu.VMEM_SHARED((NPAD, CP), jnp.float32),
            pltpu.SemaphoreType.DMA,
            pltpu.SemaphoreType.DMA,
            pltpu.SemaphoreType.DMA,
        ],
    )
    def step(pprev_hbm, upp_hbm, ee_hbm, gg_hbm, rowv_hbm, colv_hbm, zeros_hbm,
             u_out, p_out,
             row_v, col_v, msg0, msg1, p0c, p1c, uppc, eec, ggc, outc,
             u_sh, acc, semA, sem0, sem1):
        cid = lax.axis_index("c")
        sid = lax.axis_index("s")
        wid = sid * 2 + cid
        pltpu.sync_copy(rowv_hbm.at[wid], row_v)
        pltpu.sync_copy(colv_hbm.at[wid], col_v)
        pltpu.sync_copy(zeros_hbm.at[pl.ds(sid * ROWS_PT, ROWS_PT)],
                        acc.at[pl.ds(sid * ROWS_PT, ROWS_PT)])

        for t in range(NCA):
            r0 = sid * ROWS_PT + t * CA
            d0 = pltpu.async_copy(pprev_hbm.at[pl.ds(r0, CA)], p0c, semA)
            d1 = pltpu.async_copy(pprev_hbm.at[pl.ds(NPAD + r0, CA)], p1c, semA)
            d2 = pltpu.async_copy(upp_hbm.at[pl.ds(r0, CA)], uppc, semA)
            d3 = pltpu.async_copy(ee_hbm.at[pl.ds(r0, CA)], eec, semA)
            d4 = pltpu.async_copy(gg_hbm.at[pl.ds(r0, CA)], ggc, semA)
            d0.wait(); d1.wait(); d2.wait(); d3.wait(); d4.wait()

            def cbody(r, carry):
                for c in range(CP // 16):
                    s = pl.ds(16 * c, 16)
                    outc[r, s] = (eec[r, s] * (p0c[r, s] + p1c[r, s])
                                  + ggc[r, s] * uppc[r, s])
                return carry

            lax.fori_loop(0, CA, cbody, 0)
            pltpu.sync_copy(outc, u_sh.at[pl.ds(r0, CA)])

            @pl.when(cid == 0)
            def _():
                pltpu.sync_copy(outc, u_out.at[pl.ds(r0, CA)])

        plsc.subcore_barrier()

        pltpu.async_copy(u_sh.at[row_v.at[0]], msg0, sem0)

        def body(g, carry):
            j0 = 2 * g
            j1 = j0 + 1
            j2 = j0 + 2
            pltpu.async_copy(u_sh.at[row_v.at[j1]], msg1, sem1)
            pltpu.make_async_copy(u_sh.at[row_v.at[j0]], msg0, sem0).wait()
            pltpu.sync_copy(msg0, acc.at[col_v.at[j0]], add=True)

            @pl.when(j2 < NCH)
            def _():
                pltpu.async_copy(u_sh.at[row_v.at[j2]], msg0, sem0)

            pltpu.make_async_copy(u_sh.at[row_v.at[j1]], msg1, sem1).wait()
            pltpu.sync_copy(msg1, acc.at[col_v.at[j1]], add=True)
            return carry

        lax.fori_loop(0, NCH // 2, body, 0)
        plsc.subcore_barrier()
        pltpu.sync_copy(acc.at[pl.ds(sid * ROWS_PT, ROWS_PT)],
                        p_out.at[pl.ds(cid * NPAD + sid * ROWS_PT, ROWS_PT)])

    return step


def _final_body(*refs):
    (u_refs, p_ref, e_ref, g_ref, temp_ref, dinv_ref, pw_ref, pb_ref,
     o_ref) = refs[:K], refs[K], refs[K + 1], refs[K + 2], refs[K + 3], refs[K + 4], refs[K + 5], refs[K + 6], refs[K + 7]
    dinv = dinv_ref[...]
    pw = pw_ref[...]
    pb = pb_ref[0, 0]
    bn = dinv.shape[0]

    us = [r[...] for r in u_refs]
    u_last = (e_ref[...] * (p_ref[0] + p_ref[1]) + g_ref[...] * us[K - 1])
    us.append(u_last)

    out = jnp.zeros((bn, CP), jnp.float32)
    running = temp_ref[0, 0] * us[0]
    for j in range(K + 1):
        if j == 0:
            pj = dinv * us[0]
        else:
            running = running + temp_ref[0, j] * us[j]
            pj = dinv * running
        logit = jnp.sum(pj * pw, axis=-1, keepdims=True) + pb
        out = out + jax.nn.sigmoid(logit) * pj

    mask = lax.broadcasted_iota(jnp.int32, (bn, CP), 1) < C
    m = jnp.max(jnp.where(mask, out, -1e30), axis=-1, keepdims=True)
    z = out - m
    ez = jnp.where(mask, jnp.exp(z), 0.0)
    lse = jnp.log(jnp.sum(ez, axis=-1, keepdims=True))
    o_ref[...] = z - lse


def _final(us, p, e, g, temp, dinv, proj_w, proj_b):
    tempp = jnp.pad(temp.reshape(1, K + 1), ((0, 0), (0, 16 - (K + 1))))
    pwp = jnp.pad(proj_w, ((0, 0), (0, CP - C)))
    pbr = proj_b.reshape(1, 1)
    row_spec = pl.BlockSpec((_BN, CP), lambda i: (i, 0))
    col_spec = pl.BlockSpec((_BN, 1), lambda i: (i, 0))
    return pl.pallas_call(
        _final_body,
        grid=(NPAD // _BN,),
        in_specs=(
            [row_spec] * K
            + [pl.BlockSpec((2, _BN, CP), lambda i: (0, i, 0))]
            + [col_spec, col_spec,
               pl.BlockSpec((1, 16), lambda i: (0, 0)),
               col_spec,
               pl.BlockSpec((1, CP), lambda i: (0, 0)),
               pl.BlockSpec((1, 1), lambda i: (0, 0))]
        ),
        out_specs=pl.BlockSpec((_BN, CP), lambda i: (i, 0)),
        out_shape=jax.ShapeDtypeStruct((NPAD, CP), jnp.float32),
    )(*us, p.reshape(2, NPAD, CP), e, g, tempp, dinv, pwp, pbr)


def kernel(x, edge_index, W1, b1, W2, b2, proj_w, proj_b, temp):
    row = edge_index[0]
    col = edge_index[1]

    loop_mask = row == col
    loop_counts = jnp.zeros((N,), jnp.int32).at[col].add(loop_mask.astype(jnp.int32))
    self_w = jnp.where(loop_counts > 0, 0.0, 1.0).astype(jnp.float32)
    deg = jnp.zeros((N,), jnp.float32).at[col].add(jnp.ones((E,), jnp.float32)) + self_w

    zpad = jnp.zeros((NPAD - N,), jnp.float32)
    e = jnp.concatenate([1.0 / deg, zpad]).reshape(NPAD, 1)
    g = jnp.concatenate([self_w / deg, zpad]).reshape(NPAD, 1)
    d = jnp.concatenate([deg ** -0.5, zpad]).reshape(NPAD, 1)
    dinv = jnp.concatenate([deg ** 0.5, zpad]).reshape(NPAD, 1)

    pad_idx = (N + (jnp.arange(EPAD - E, dtype=jnp.int32) % (NPAD - N))).astype(jnp.int32)
    rowp = jnp.concatenate([row, pad_idx]).reshape(NTILES, NCH, CH)
    colp = jnp.concatenate([col, pad_idx]).reshape(NTILES, NCH, CH)
    zeros_nc = jnp.zeros((NPAD, CP), jnp.float32)

    xp = jnp.pad(x, ((0, NPAD - N), (0, 0)))
    u0 = _mlp(xp, W1, b1, W2, b2, d)

    ee = jnp.broadcast_to(e, (NPAD, CP))
    gg = jnp.broadcast_to(g, (NPAD, CP))

    sc_step = _make_sc_step()
    sc_fused = _make_sc_fused_step()

    us = [u0]
    p = sc_step(u0, rowp, colp, zeros_nc)
    u_pp = u0
    for _ in range(K - 1):
        u_new, p = sc_fused(p, u_pp, ee, gg, rowp, colp, zeros_nc)
        us.append(u_new)
        u_pp = u_new

    out = _final(us, p, e, g, temp, dinv, proj_w, proj_b)
    return out[:N, :C]
